# baseline (speedup 1.0000x reference)
import jax
import jax.numpy as jnp
from jax.experimental import pallas as pl
from jax.experimental.pallas import tpu as pltpu

_CHUNK_ROWS = 10000
_K = 8
_P = 3


def _chunks_for(n):
    out = []
    off = 0
    while off < n:
        out.append((off, min(_CHUNK_ROWS, n - off)))
        off += _CHUNK_ROWS
    return out


def kernel(embed_user, embed_item):
    n_u, e = embed_user.shape
    n_i, _ = embed_item.shape

    def body(u_ref, i_ref, ou_ref, oi_ref, bufs, sem_in, sem_out):
        chunks = (
            [(u_ref, ou_ref, off, sz) for off, sz in _chunks_for(n_u)] +
            [(i_ref, oi_ref, off, sz) for off, sz in _chunks_for(n_i)]
        )
        t_total = len(chunks)

        def fill(t):
            src, _, off, sz = chunks[t]
            slot = t % _K
            pltpu.make_async_copy(
                src.at[pl.ds(off, sz)], bufs.at[slot, pl.ds(0, sz)],
                sem_in.at[slot]).start()

        def drain_started(t):
            _, dst, off, sz = chunks[t]
            slot = t % _K
            return pltpu.make_async_copy(
                bufs.at[slot, pl.ds(0, sz)], dst.at[pl.ds(off, sz)],
                sem_out.at[slot])

        lookahead = min(_P, t_total)
        for t in range(lookahead):
            fill(t)
        for t in range(t_total):
            src, dst, off, sz = chunks[t]
            slot = t % _K
            pltpu.make_async_copy(
                src.at[pl.ds(off, sz)], bufs.at[slot, pl.ds(0, sz)],
                sem_in.at[slot]).wait()
            drain_started(t).start()
            nt = t + lookahead
            if nt < t_total:
                if nt >= _K:
                    drain_started(nt - _K).wait()
                fill(nt)
        for t in range(max(0, t_total - _K), t_total):
            drain_started(t).wait()

    hbm = pl.BlockSpec(memory_space=pltpu.MemorySpace.HBM)
    out_u, out_i = pl.pallas_call(
        body,
        in_specs=[hbm, hbm],
        out_specs=[hbm, hbm],
        out_shape=[
            jax.ShapeDtypeStruct((n_u, e), embed_user.dtype),
            jax.ShapeDtypeStruct((n_i, e), embed_item.dtype),
        ],
        scratch_shapes=[
            pltpu.VMEM((_K, _CHUNK_ROWS, e), embed_user.dtype),
            pltpu.SemaphoreType.DMA((_K,)),
            pltpu.SemaphoreType.DMA((_K,)),
        ],
    )(embed_user, embed_item)
    return (out_u, out_i)


# speedup vs baseline: 1.0051x; 1.0051x over previous
import jax
import jax.numpy as jnp
from jax.experimental import pallas as pl
from jax.experimental.pallas import tpu as pltpu

_CHUNK_ROWS = 20000
_K = 5
_P = 2


def _chunks_for(n):
    out = []
    off = 0
    while off < n:
        out.append((off, min(_CHUNK_ROWS, n - off)))
        off += _CHUNK_ROWS
    return out


def kernel(embed_user, embed_item):
    n_u, e = embed_user.shape
    n_i, _ = embed_item.shape

    def body(u_ref, i_ref, ou_ref, oi_ref, bufs, sem_in, sem_out):
        chunks = (
            [(u_ref, ou_ref, off, sz) for off, sz in _chunks_for(n_u)] +
            [(i_ref, oi_ref, off, sz) for off, sz in _chunks_for(n_i)]
        )
        t_total = len(chunks)

        def fill(t):
            src, _, off, sz = chunks[t]
            slot = t % _K
            pltpu.make_async_copy(
                src.at[pl.ds(off, sz)], bufs.at[slot, pl.ds(0, sz)],
                sem_in.at[slot]).start()

        def drain_started(t):
            _, dst, off, sz = chunks[t]
            slot = t % _K
            return pltpu.make_async_copy(
                bufs.at[slot, pl.ds(0, sz)], dst.at[pl.ds(off, sz)],
                sem_out.at[slot])

        lookahead = min(_P, t_total)
        for t in range(lookahead):
            fill(t)
        for t in range(t_total):
            src, dst, off, sz = chunks[t]
            slot = t % _K
            pltpu.make_async_copy(
                src.at[pl.ds(off, sz)], bufs.at[slot, pl.ds(0, sz)],
                sem_in.at[slot]).wait()
            drain_started(t).start()
            nt = t + lookahead
            if nt < t_total:
                if nt >= _K:
                    drain_started(nt - _K).wait()
                fill(nt)
        for t in range(max(0, t_total - _K), t_total):
            drain_started(t).wait()

    hbm = pl.BlockSpec(memory_space=pltpu.MemorySpace.HBM)
    out_u, out_i = pl.pallas_call(
        body,
        in_specs=[hbm, hbm],
        out_specs=[hbm, hbm],
        out_shape=[
            jax.ShapeDtypeStruct((n_u, e), embed_user.dtype),
            jax.ShapeDtypeStruct((n_i, e), embed_item.dtype),
        ],
        scratch_shapes=[
            pltpu.VMEM((_K, _CHUNK_ROWS, e), embed_user.dtype),
            pltpu.SemaphoreType.DMA((_K,)),
            pltpu.SemaphoreType.DMA((_K,)),
        ],
    )(embed_user, embed_item)
    return (out_u, out_i)
